# baseline (speedup 1.0000x reference)
import jax
import jax.numpy as jnp
from jax import lax
from jax.experimental import pallas as pl

GAMMA_ = 2.0
ALPHA_ = 0.25


def _focal_loss_kernel(x_ref, t_ref, w_ref, o_ref):
    x = x_ref[...]
    t = t_ref[...]
    w = w_ref[...]
    idx = lax.broadcasted_iota(jnp.int32, x.shape, 1) // 3
    w3 = jnp.take_along_axis(w, idx, axis=1)

    absx = jnp.abs(x)
    e = jnp.exp(-absx)
    sig = jnp.where(x >= 0.0, 1.0, e) / (1.0 + e)
    pt = t + sig - 2.0 * t * sig
    alpha_w = (1.0 - ALPHA_) - (1.0 - 2.0 * ALPHA_) * t
    bce = jnp.maximum(x, 0.0) - x * t + jnp.log1p(e)
    o_ref[...] = (alpha_w * (pt * pt)) * (bce * w3)


def kernel(input, target, weights, groups, epoch):
    B, A, C = input.shape
    rows = (B * A) // 128
    lanes = 128 * C
    x2 = input.reshape(rows, lanes)
    t2 = target.reshape(rows, lanes)
    w2 = weights.reshape(rows, 128)

    R = 1024
    grid = (rows // R,)
    out = pl.pallas_call(
        _focal_loss_kernel,
        out_shape=jax.ShapeDtypeStruct((rows, lanes), jnp.float32),
        grid=grid,
        in_specs=[
            pl.BlockSpec((R, lanes), lambda i: (i, 0)),
            pl.BlockSpec((R, lanes), lambda i: (i, 0)),
            pl.BlockSpec((R, 128), lambda i: (i, 0)),
        ],
        out_specs=pl.BlockSpec((R, lanes), lambda i: (i, 0)),
    )(x2, t2, w2)
    return out.reshape(B, A, C)


# speedup vs baseline: 5.2067x; 5.2067x over previous
import jax
import jax.numpy as jnp
from jax import lax
from jax.experimental import pallas as pl

GAMMA_ = 2.0
ALPHA_ = 0.25


def _focal_loss_kernel(x_ref, t_ref, w_ref, o_ref):
    x = x_ref[...]
    t = t_ref[...]
    w = w_ref[...]
    nchunk = w.shape[1] // 128
    idx = lax.broadcasted_iota(jnp.int32, (w.shape[0], 384), 1) // 3
    w3 = jnp.concatenate(
        [jnp.take_along_axis(w[:, 128 * k:128 * (k + 1)], idx, axis=1)
         for k in range(nchunk)], axis=1)

    absx = jnp.abs(x)
    e = jnp.exp(-absx)
    u = 1.0 / (1.0 + e)
    sig = jnp.where(x >= 0.0, u, 1.0 - u)
    ts = t * sig
    pt = (sig - ts) + (t - ts)
    alpha_w = 0.75 - 0.5 * t
    bce = (jnp.maximum(x, 0.0) - x * t) + jnp.log1p(e)
    o_ref[...] = (alpha_w * (pt * pt)) * (bce * w3)


def kernel(input, target, weights, groups, epoch):
    B, A, C = input.shape
    x2 = input.reshape(B, A * C)
    t2 = target.reshape(B, A * C)

    LB = 12288
    grid = ((A * C) // LB,)
    out = pl.pallas_call(
        _focal_loss_kernel,
        out_shape=jax.ShapeDtypeStruct((B, A * C), jnp.float32),
        grid=grid,
        in_specs=[
            pl.BlockSpec((B, LB), lambda i: (0, i)),
            pl.BlockSpec((B, LB), lambda i: (0, i)),
            pl.BlockSpec((B, LB // 3), lambda i: (0, i)),
        ],
        out_specs=pl.BlockSpec((B, LB), lambda i: (0, i)),
    )(x2, t2, weights)
    return out.reshape(B, A, C)


# speedup vs baseline: 379.5623x; 72.8992x over previous
import jax
import jax.numpy as jnp
from jax.experimental import pallas as pl

GAMMA_ = 2.0
ALPHA_ = 0.25


def _focal_loss_kernel(x_ref, t_ref, w_ref, o_ref):
    x = x_ref[...]
    t = t_ref[...]
    w = w_ref[...]

    absx = jnp.abs(x)
    e = jnp.exp(-absx)
    u = 1.0 / (1.0 + e)
    sig = jnp.where(x >= 0.0, u, 1.0 - u)
    ts = t * sig
    pt = (sig - ts) + (t - ts)
    alpha_w = 0.75 - 0.5 * t
    bce = (jnp.maximum(x, 0.0) - x * t) + jnp.log1p(e)
    o_ref[...] = (alpha_w * (pt * pt)) * (bce * w[None])


def kernel(input, target, weights, groups, epoch):
    B, A, C = input.shape
    xt = jnp.transpose(input, (2, 0, 1))
    tt = jnp.transpose(target, (2, 0, 1))

    LB = 16384
    grid = (A // LB,)
    out = pl.pallas_call(
        _focal_loss_kernel,
        out_shape=jax.ShapeDtypeStruct((C, B, A), jnp.float32),
        grid=grid,
        in_specs=[
            pl.BlockSpec((C, B, LB), lambda i: (0, 0, i)),
            pl.BlockSpec((C, B, LB), lambda i: (0, 0, i)),
            pl.BlockSpec((B, LB), lambda i: (0, i)),
        ],
        out_specs=pl.BlockSpec((C, B, LB), lambda i: (0, 0, i)),
    )(xt, tt, weights)
    return jnp.transpose(out, (1, 2, 0))


# speedup vs baseline: 382.7310x; 1.0083x over previous
import jax
import jax.numpy as jnp
from jax.experimental import pallas as pl
from jax.experimental.pallas import tpu as pltpu

GAMMA_ = 2.0
ALPHA_ = 0.25


def _focal_loss_kernel(x_ref, t_ref, w_ref, o_ref):
    x = x_ref[...]
    t = t_ref[...]
    w = w_ref[...]

    absx = jnp.abs(x)
    e = jnp.exp(-absx)
    u = 1.0 / (1.0 + e)
    sig = jnp.where(x >= 0.0, u, 1.0 - u)
    ts = t * sig
    pt = (sig - ts) + (t - ts)
    alpha_w = 0.75 - 0.5 * t
    bce = (jnp.maximum(x, 0.0) - x * t) + jnp.log1p(e)
    o_ref[...] = (alpha_w * (pt * pt)) * (bce * w)


def kernel(input, target, weights, groups, epoch):
    B, A, C = input.shape
    xt = jnp.transpose(input, (2, 0, 1))
    tt = jnp.transpose(target, (2, 0, 1))
    wt = weights[None]

    LB = 16384
    grid = (A // LB,)
    out = pl.pallas_call(
        _focal_loss_kernel,
        out_shape=jax.ShapeDtypeStruct((C, B, A), jnp.float32),
        grid=grid,
        in_specs=[
            pl.BlockSpec((C, B, LB), lambda i: (0, 0, i)),
            pl.BlockSpec((C, B, LB), lambda i: (0, 0, i)),
            pl.BlockSpec((1, B, LB), lambda i: (0, 0, i)),
        ],
        out_specs=pl.BlockSpec((C, B, LB), lambda i: (0, 0, i)),
        compiler_params=pltpu.CompilerParams(
            dimension_semantics=("parallel",),
        ),
    )(xt, tt, wt)
    return jnp.transpose(out, (1, 2, 0))


# speedup vs baseline: 418.3671x; 1.0931x over previous
import jax
import jax.numpy as jnp
from jax.experimental import pallas as pl
from jax.experimental.pallas import tpu as pltpu

GAMMA_ = 2.0
ALPHA_ = 0.25

LB_ = 32768


def _focal_loss_kernel(x_ref, t_ref, w_ref, o_ref):
    x = x_ref[...]
    t = t_ref[...]
    w = w_ref[...]

    absx = jnp.abs(x)
    e = jnp.exp(-absx)
    u = 1.0 / (1.0 + e)
    sig = jnp.where(x >= 0.0, u, 1.0 - u)
    ts = t * sig
    pt = (sig - ts) + (t - ts)
    alpha_w = 0.75 - 0.5 * t
    bce = (jnp.maximum(x, 0.0) - x * t) + jnp.log1p(e)
    o_ref[...] = (alpha_w * (pt * pt)) * (bce * w)


def kernel(input, target, weights, groups, epoch):
    B, A, C = input.shape
    xt = jnp.transpose(input, (2, 0, 1))
    tt = jnp.transpose(target, (2, 0, 1))
    wt = weights[None]

    grid = (A // LB_,)
    out = pl.pallas_call(
        _focal_loss_kernel,
        out_shape=jax.ShapeDtypeStruct((C, B, A), jnp.float32),
        grid=grid,
        in_specs=[
            pl.BlockSpec((C, B, LB_), lambda i: (0, 0, i)),
            pl.BlockSpec((C, B, LB_), lambda i: (0, 0, i)),
            pl.BlockSpec((1, B, LB_), lambda i: (0, 0, i)),
        ],
        out_specs=pl.BlockSpec((C, B, LB_), lambda i: (0, 0, i)),
        compiler_params=pltpu.CompilerParams(
            dimension_semantics=("parallel",),
        ),
    )(xt, tt, wt)
    return jnp.transpose(out, (1, 2, 0))


# speedup vs baseline: 459.1613x; 1.0975x over previous
import jax
import jax.numpy as jnp
from jax.experimental import pallas as pl
from jax.experimental.pallas import tpu as pltpu

GAMMA_ = 2.0
ALPHA_ = 0.25

LB_ = 32768


def _focal_loss_kernel(x_ref, t_ref, w_ref, o_ref):
    x = x_ref[...]
    t = t_ref[...]
    w = w_ref[...]

    th = jnp.tanh(x * 0.5)
    sig = 0.5 + 0.5 * th
    q = sig - th
    bce = jnp.log(q) * -1.0 - x * t
    ts = t * sig
    pt = (sig - ts) + (t - ts)
    alpha_w = 0.75 - 0.5 * t
    o_ref[...] = (alpha_w * (pt * pt)) * (bce * w)


def kernel(input, target, weights, groups, epoch):
    B, A, C = input.shape
    xt = jnp.transpose(input, (2, 0, 1))
    tt = jnp.transpose(target, (2, 0, 1))
    wt = weights[None]

    grid = (A // LB_,)
    out = pl.pallas_call(
        _focal_loss_kernel,
        out_shape=jax.ShapeDtypeStruct((C, B, A), jnp.float32),
        grid=grid,
        in_specs=[
            pl.BlockSpec((C, B, LB_), lambda i: (0, 0, i)),
            pl.BlockSpec((C, B, LB_), lambda i: (0, 0, i)),
            pl.BlockSpec((1, B, LB_), lambda i: (0, 0, i)),
        ],
        out_specs=pl.BlockSpec((C, B, LB_), lambda i: (0, 0, i)),
        compiler_params=pltpu.CompilerParams(
            dimension_semantics=("parallel",),
        ),
    )(xt, tt, wt)
    return jnp.transpose(out, (1, 2, 0))
